# baseline (speedup 1.0000x reference)
import functools

import jax
import jax.numpy as jnp
from jax import lax
from jax.experimental import pallas as pl
from jax.experimental.pallas import tpu as pltpu
from jax.experimental.pallas import tpu_sc as plsc

NC = 2
NS = 16
CCH = 125
BN = 1000


def _hw_body(h_ref, wd_ref, out_ref):
    out_ref[...] = jnp.dot(h_ref[...], wd_ref[...],
                           preferred_element_type=jnp.float32)


def _tc_hw(h, wd):
    n, hdim = h.shape
    return pl.pallas_call(
        _hw_body,
        grid=(n // BN,),
        in_specs=[
            pl.BlockSpec((BN, hdim), lambda i: (i, 0)),
            pl.BlockSpec((hdim, hdim), lambda i: (0, 0)),
        ],
        out_specs=pl.BlockSpec((BN, hdim), lambda i: (i, 0)),
        out_shape=jax.ShapeDtypeStruct((n, hdim), jnp.float32),
    )(h, wd)


def _update_body(h_ref, s2_ref, da_ref, wsrc_ref, wattr_ref, bmsg_ref,
                 wih_ref, whh_ref, bih_ref, bhh_ref, wdn_ref,
                 hout_ref, hwout_ref):
    hdim = h_ref.shape[1]
    h = h_ref[...]
    s = s2_ref[0] + s2_ref[1]
    deg = da_ref[0, :, 16:17] + da_ref[1, :, 16:17]
    attr_sum = da_ref[0, :, 0:1] + da_ref[1, :, 0:1]
    agg = (s
           + deg * jnp.dot(h, wsrc_ref[...], preferred_element_type=jnp.float32)
           + attr_sum * wattr_ref[...]
           + deg * bmsg_ref[...])
    gi = jnp.dot(agg, wih_ref[...], preferred_element_type=jnp.float32) + bih_ref[...]
    gh = jnp.dot(h, whh_ref[...], preferred_element_type=jnp.float32) + bhh_ref[...]
    r = jax.nn.sigmoid(gi[:, :hdim] + gh[:, :hdim])
    z = jax.nn.sigmoid(gi[:, hdim:2 * hdim] + gh[:, hdim:2 * hdim])
    nn = jnp.tanh(gi[:, 2 * hdim:] + r * gh[:, 2 * hdim:])
    hn = (1.0 - z) * nn + z * h
    hout_ref[...] = hn
    hwout_ref[...] = jnp.dot(hn, wdn_ref[...], preferred_element_type=jnp.float32)


def _tc_update(h, s2, da2, wsrc, wattr, bmsg, wih, whh, bih, bhh, wdn):
    n, hdim = h.shape
    full = lambda a, b: pl.BlockSpec((a, b), lambda i: (0, 0))
    return pl.pallas_call(
        _update_body,
        grid=(n // BN,),
        in_specs=[
            pl.BlockSpec((BN, hdim), lambda i: (i, 0)),
            pl.BlockSpec((2, BN, hdim), lambda i: (0, i, 0)),
            pl.BlockSpec((2, BN, hdim), lambda i: (0, i, 0)),
            full(hdim, hdim),
            full(1, hdim),
            full(1, hdim),
            full(hdim, 3 * hdim),
            full(hdim, 3 * hdim),
            full(1, 3 * hdim),
            full(1, 3 * hdim),
            full(hdim, hdim),
        ],
        out_specs=[
            pl.BlockSpec((BN, hdim), lambda i: (i, 0)),
            pl.BlockSpec((BN, hdim), lambda i: (i, 0)),
        ],
        out_shape=[
            jax.ShapeDtypeStruct((n, hdim), jnp.float32),
            jax.ShapeDtypeStruct((n, hdim), jnp.float32),
        ],
    )(h, s2, da2, wsrc, wattr, bmsg, wih, whh, bih, bhh, wdn)


def _sc_mesh():
    return plsc.VectorSubcoreMesh(
        core_axis_name="c", subcore_axis_name="s", num_cores=NC, num_subcores=NS)


def _row_split(n_nodes):
    rpt = -(-(n_nodes // NS) // 8) * 8
    rlast = n_nodes - (NS - 1) * rpt
    return rpt, rlast


def _make_sc_da(n_nodes, n_edges, hdim):
    epw = n_edges // (NC * NS)
    nch = epw // CCH
    rpt, rlast = _row_split(n_nodes)
    nvec = -(-CCH // 16)

    @functools.partial(
        pl.kernel,
        out_type=jax.ShapeDtypeStruct((NC * n_nodes, hdim), jnp.float32),
        mesh=_sc_mesh(),
        scratch_types=[
            pltpu.VMEM((CCH,), jnp.int32),
            pltpu.VMEM((CCH,), jnp.int32),
            pltpu.VMEM((nvec * 16,), jnp.float32),
            pltpu.VMEM((nvec * 16,), jnp.float32),
            pltpu.VMEM((nvec * 16, hdim), jnp.float32),
            pltpu.VMEM((nvec * 16, hdim), jnp.float32),
            pltpu.VMEM_SHARED((n_nodes, hdim), jnp.float32),
            pltpu.SemaphoreType.DMA,
            pltpu.SemaphoreType.DMA,
            pltpu.SemaphoreType.DMA,
            pltpu.SemaphoreType.DMA,
        ],
    )
    def k(row2_hbm, attr2_hbm, zeros_hbm, out_hbm,
          rva, rvb, aba, abb, vba, vbb, dacc, asem, rsem, ssema, ssemb):
        c = lax.axis_index("c")
        s = lax.axis_index("s")
        tid = c * NS + s
        r0 = pl.multiple_of(s * rpt, 8)
        ob = pl.multiple_of(c * n_nodes + s * rpt, 8)
        ch0 = tid * nch

        @pl.when(s < NS - 1)
        def _():
            pltpu.sync_copy(zeros_hbm.at[pl.ds(r0, rpt)], dacc.at[pl.ds(r0, rpt)])

        @pl.when(s == NS - 1)
        def _():
            pltpu.sync_copy(zeros_hbm.at[pl.ds((NS - 1) * rpt, rlast)],
                            dacc.at[pl.ds((NS - 1) * rpt, rlast)])

        lanes = lax.iota(jnp.int32, 16)
        onevec = jnp.where(lanes == 0, 1.0, 0.0).astype(jnp.float32)
        pltpu.sync_copy(zeros_hbm.at[pl.ds(0, nvec * 16)], vba)
        pltpu.sync_copy(zeros_hbm.at[pl.ds(0, nvec * 16)], vbb)
        for i in range(CCH):
            vba[i, pl.ds(16, 16)] = onevec
            vbb[i, pl.ds(16, 16)] = onevec
        plsc.subcore_barrier()

        def fill_attr(ab, vb):
            for q in range(nvec):
                av = ab[pl.ds(16 * q, 16)]
                for kk in range(min(16, CCH - 16 * q)):
                    vb[16 * q + kk, pl.ds(0, 16)] = jnp.full(
                        (16,), av[kk], jnp.float32)

        def body(p, carry):
            j0 = 2 * p
            j1 = 2 * p + 1

            @pl.when(p > 0)
            def _():
                pltpu.make_async_copy(vba.at[pl.ds(0, CCH)], dacc.at[rva],
                                      ssema).wait()

            pltpu.async_copy(row2_hbm.at[ch0 + j0], rva, rsem)
            pltpu.async_copy(attr2_hbm.at[ch0 + j0], aba, asem).wait()
            fill_attr(aba, vba)
            pltpu.make_async_copy(row2_hbm.at[ch0 + j0], rva, rsem).wait()
            pltpu.async_copy(vba.at[pl.ds(0, CCH)], dacc.at[rva], ssema,
                             add=True)

            @pl.when(p > 0)
            def _():
                pltpu.make_async_copy(vbb.at[pl.ds(0, CCH)], dacc.at[rvb],
                                      ssemb).wait()

            pltpu.async_copy(row2_hbm.at[ch0 + j1], rvb, rsem)
            pltpu.async_copy(attr2_hbm.at[ch0 + j1], abb, asem).wait()
            fill_attr(abb, vbb)
            pltpu.make_async_copy(row2_hbm.at[ch0 + j1], rvb, rsem).wait()
            pltpu.async_copy(vbb.at[pl.ds(0, CCH)], dacc.at[rvb], ssemb,
                             add=True)
            return carry

        lax.fori_loop(0, nch // 2, body, 0)
        pltpu.make_async_copy(vba.at[pl.ds(0, CCH)], dacc.at[rva], ssema).wait()
        pltpu.make_async_copy(vbb.at[pl.ds(0, CCH)], dacc.at[rvb], ssemb).wait()
        plsc.subcore_barrier()

        @pl.when(s < NS - 1)
        def _():
            pltpu.sync_copy(dacc.at[pl.ds(r0, rpt)], out_hbm.at[pl.ds(ob, rpt)])

        @pl.when(s == NS - 1)
        def _():
            pltpu.sync_copy(
                dacc.at[pl.ds((NS - 1) * rpt, rlast)],
                out_hbm.at[pl.ds(c * n_nodes + (NS - 1) * rpt, rlast)])

    return k


def _make_sc_spmm(n_nodes, n_edges, hdim):
    epw = n_edges // (NC * NS)
    nch = epw // CCH
    rpt, rlast = _row_split(n_nodes)

    @functools.partial(
        pl.kernel,
        out_type=jax.ShapeDtypeStruct((NC * n_nodes, hdim), jnp.float32),
        mesh=_sc_mesh(),
        scratch_types=[
            pltpu.VMEM((nch, CCH), jnp.int32),
            pltpu.VMEM((CCH,), jnp.int32),
            pltpu.VMEM((CCH,), jnp.int32),
            pltpu.VMEM((CCH, hdim), jnp.float32),
            pltpu.VMEM((CCH, hdim), jnp.float32),
            pltpu.VMEM_SHARED((n_nodes, hdim), jnp.float32),
            pltpu.SemaphoreType.DMA,
            pltpu.SemaphoreType.DMA,
            pltpu.SemaphoreType.DMA,
            pltpu.SemaphoreType.DMA,
            pltpu.SemaphoreType.DMA,
            pltpu.SemaphoreType.DMA,
        ],
    )
    def k(col2_hbm, row2_hbm, hw_hbm, zeros_hbm, out_hbm,
          col_all, rva, rvb, bufa, bufb, sacc,
          gsema, gsemb, rsem, rsemb, ssema, ssemb):
        c = lax.axis_index("c")
        s = lax.axis_index("s")
        tid = c * NS + s
        r0 = pl.multiple_of(s * rpt, 8)
        ob = pl.multiple_of(c * n_nodes + s * rpt, 8)
        ch0 = tid * nch
        pltpu.sync_copy(col2_hbm.at[pl.ds(ch0, nch)], col_all)

        @pl.when(s < NS - 1)
        def _():
            pltpu.sync_copy(zeros_hbm.at[pl.ds(r0, rpt)], sacc.at[pl.ds(r0, rpt)])

        @pl.when(s == NS - 1)
        def _():
            pltpu.sync_copy(zeros_hbm.at[pl.ds((NS - 1) * rpt, rlast)],
                            sacc.at[pl.ds((NS - 1) * rpt, rlast)])

        plsc.subcore_barrier()

        def body(p, carry):
            j0 = 2 * p
            j1 = 2 * p + 1

            @pl.when(p > 0)
            def _():
                pltpu.make_async_copy(bufa, sacc.at[rva], ssema).wait()

            pltpu.async_copy(row2_hbm.at[ch0 + j0], rva, rsem)
            pltpu.async_copy(hw_hbm.at[col_all.at[j0]], bufa, gsema)

            @pl.when(p > 0)
            def _():
                pltpu.make_async_copy(bufb, sacc.at[rvb], ssemb).wait()

            pltpu.async_copy(row2_hbm.at[ch0 + j1], rvb, rsemb)
            pltpu.async_copy(hw_hbm.at[col_all.at[j1]], bufb, gsemb)

            pltpu.make_async_copy(row2_hbm.at[ch0 + j0], rva, rsem).wait()
            pltpu.make_async_copy(hw_hbm.at[col_all.at[j0]], bufa, gsema).wait()
            pltpu.async_copy(bufa, sacc.at[rva], ssema, add=True)

            pltpu.make_async_copy(row2_hbm.at[ch0 + j1], rvb, rsemb).wait()
            pltpu.make_async_copy(hw_hbm.at[col_all.at[j1]], bufb, gsemb).wait()
            pltpu.async_copy(bufb, sacc.at[rvb], ssemb, add=True)
            return carry

        lax.fori_loop(0, nch // 2, body, 0)
        pltpu.make_async_copy(bufa, sacc.at[rva], ssema).wait()
        pltpu.make_async_copy(bufb, sacc.at[rvb], ssemb).wait()
        plsc.subcore_barrier()

        @pl.when(s < NS - 1)
        def _():
            pltpu.sync_copy(sacc.at[pl.ds(r0, rpt)], out_hbm.at[pl.ds(ob, rpt)])

        @pl.when(s == NS - 1)
        def _():
            pltpu.sync_copy(
                sacc.at[pl.ds((NS - 1) * rpt, rlast)],
                out_hbm.at[pl.ds(c * n_nodes + (NS - 1) * rpt, rlast)])

    return k


def kernel(x, edge_index, edge_attr, W_msg, b_msg, W_ih, W_hh, b_ih, b_hh):
    n, hdim = x.shape
    n_edges = edge_index.shape[1]
    rounds = W_msg.shape[0]

    nch_all = n_edges // CCH
    row2 = edge_index[0].reshape(nch_all, CCH)
    col2 = edge_index[1].reshape(nch_all, CCH)
    attr2 = jnp.concatenate(
        [edge_attr.reshape(nch_all, CCH),
         jnp.zeros((nch_all, 3), jnp.float32)], axis=1)
    zeros = jnp.zeros((n, hdim), jnp.float32)

    sc_da = _make_sc_da(n, n_edges, hdim)
    sc_spmm = _make_sc_spmm(n, n_edges, hdim)

    da2 = sc_da(row2, attr2, zeros).reshape(NC, n, hdim)

    h = x
    hw = _tc_hw(h, W_msg[0, :hdim, :])
    for t in range(rounds):
        s2 = sc_spmm(col2, row2, hw, zeros).reshape(NC, n, hdim)
        tn = (t + 1) % rounds
        h, hw = _tc_update(
            h, s2, da2,
            W_msg[t, hdim:2 * hdim, :],
            W_msg[t, 2 * hdim:2 * hdim + 1, :],
            b_msg[t:t + 1],
            W_ih[t], W_hh[t], b_ih[t:t + 1], b_hh[t:t + 1],
            W_msg[tn, :hdim, :])
    return h


# speedup vs baseline: 1.0814x; 1.0814x over previous
import functools

import jax
import jax.numpy as jnp
from jax import lax
from jax.experimental import pallas as pl
from jax.experimental.pallas import tpu as pltpu
from jax.experimental.pallas import tpu_sc as plsc

NC = 2
NS = 16
CCH = 125
BN = 2000


def _hw_body(h_ref, wd_ref, out_ref):
    out_ref[...] = jnp.dot(h_ref[...], wd_ref[...],
                           preferred_element_type=jnp.float32)


def _tc_hw(h, wd):
    n, hdim = h.shape
    return pl.pallas_call(
        _hw_body,
        grid=(n // BN,),
        in_specs=[
            pl.BlockSpec((BN, hdim), lambda i: (i, 0)),
            pl.BlockSpec((hdim, hdim), lambda i: (0, 0)),
        ],
        out_specs=pl.BlockSpec((BN, hdim), lambda i: (i, 0)),
        out_shape=jax.ShapeDtypeStruct((n, hdim), jnp.float32),
    )(h, wd)


def _update_body(h_ref, s2_ref, da_ref, wsrc_ref, wattr_ref, bmsg_ref,
                 wih_ref, whh_ref, bih_ref, bhh_ref, wdn_ref,
                 hout_ref, hwout_ref=None):
    hdim = h_ref.shape[1]
    h = h_ref[...]
    s = s2_ref[0] + s2_ref[1]
    deg = da_ref[0, :, 16:17] + da_ref[1, :, 16:17]
    attr_sum = da_ref[0, :, 0:1] + da_ref[1, :, 0:1]
    agg = (s
           + deg * jnp.dot(h, wsrc_ref[...], preferred_element_type=jnp.float32)
           + attr_sum * wattr_ref[...]
           + deg * bmsg_ref[...])
    gi = jnp.dot(agg, wih_ref[...], preferred_element_type=jnp.float32) + bih_ref[...]
    gh = jnp.dot(h, whh_ref[...], preferred_element_type=jnp.float32) + bhh_ref[...]
    r = jax.nn.sigmoid(gi[:, :hdim] + gh[:, :hdim])
    z = jax.nn.sigmoid(gi[:, hdim:2 * hdim] + gh[:, hdim:2 * hdim])
    nn = jnp.tanh(gi[:, 2 * hdim:] + r * gh[:, 2 * hdim:])
    hn = (1.0 - z) * nn + z * h
    hout_ref[...] = hn
    if hwout_ref is not None:
        hwout_ref[...] = jnp.dot(hn, wdn_ref[...],
                                 preferred_element_type=jnp.float32)


def _tc_update(h, s2, da2, wsrc, wattr, bmsg, wih, whh, bih, bhh, wdn,
               emit_hw=True):
    n, hdim = h.shape
    full = lambda a, b: pl.BlockSpec((a, b), lambda i: (0, 0))
    out_specs = [pl.BlockSpec((BN, hdim), lambda i: (i, 0))]
    out_shape = [jax.ShapeDtypeStruct((n, hdim), jnp.float32)]
    if emit_hw:
        out_specs.append(pl.BlockSpec((BN, hdim), lambda i: (i, 0)))
        out_shape.append(jax.ShapeDtypeStruct((n, hdim), jnp.float32))
    return pl.pallas_call(
        _update_body,
        grid=(n // BN,),
        in_specs=[
            pl.BlockSpec((BN, hdim), lambda i: (i, 0)),
            pl.BlockSpec((2, BN, hdim), lambda i: (0, i, 0)),
            pl.BlockSpec((2, BN, hdim), lambda i: (0, i, 0)),
            full(hdim, hdim),
            full(1, hdim),
            full(1, hdim),
            full(hdim, 3 * hdim),
            full(hdim, 3 * hdim),
            full(1, 3 * hdim),
            full(1, 3 * hdim),
            full(hdim, hdim),
        ],
        out_specs=out_specs,
        out_shape=out_shape,
    )(h, s2, da2, wsrc, wattr, bmsg, wih, whh, bih, bhh, wdn)


def _sc_mesh():
    return plsc.VectorSubcoreMesh(
        core_axis_name="c", subcore_axis_name="s", num_cores=NC, num_subcores=NS)


def _row_split(n_nodes):
    rpt = -(-(n_nodes // NS) // 8) * 8
    rlast = n_nodes - (NS - 1) * rpt
    return rpt, rlast


def _make_sc_da(n_nodes, n_edges, hdim):
    epw = n_edges // (NC * NS)
    nch = epw // CCH
    rpt, rlast = _row_split(n_nodes)
    nvec = -(-CCH // 16)

    @functools.partial(
        pl.kernel,
        out_type=jax.ShapeDtypeStruct((NC * n_nodes, hdim), jnp.float32),
        mesh=_sc_mesh(),
        scratch_types=[
            pltpu.VMEM((CCH,), jnp.int32),
            pltpu.VMEM((CCH,), jnp.int32),
            pltpu.VMEM((nvec * 16,), jnp.float32),
            pltpu.VMEM((nvec * 16,), jnp.float32),
            pltpu.VMEM((nvec * 16, hdim), jnp.float32),
            pltpu.VMEM((nvec * 16, hdim), jnp.float32),
            pltpu.VMEM_SHARED((n_nodes, hdim), jnp.float32),
            pltpu.SemaphoreType.DMA,
            pltpu.SemaphoreType.DMA,
            pltpu.SemaphoreType.DMA,
            pltpu.SemaphoreType.DMA,
        ],
    )
    def k(row2_hbm, attr2_hbm, zeros_hbm, out_hbm,
          rva, rvb, aba, abb, vba, vbb, dacc, asem, rsem, ssema, ssemb):
        c = lax.axis_index("c")
        s = lax.axis_index("s")
        tid = c * NS + s
        r0 = pl.multiple_of(s * rpt, 8)
        ob = pl.multiple_of(c * n_nodes + s * rpt, 8)
        ch0 = tid * nch

        @pl.when(s < NS - 1)
        def _():
            pltpu.sync_copy(zeros_hbm.at[pl.ds(r0, rpt)], dacc.at[pl.ds(r0, rpt)])

        @pl.when(s == NS - 1)
        def _():
            pltpu.sync_copy(zeros_hbm.at[pl.ds((NS - 1) * rpt, rlast)],
                            dacc.at[pl.ds((NS - 1) * rpt, rlast)])

        lanes = lax.iota(jnp.int32, 16)
        onevec = jnp.where(lanes == 0, 1.0, 0.0).astype(jnp.float32)
        pltpu.sync_copy(zeros_hbm.at[pl.ds(0, nvec * 16)], vba)
        pltpu.sync_copy(zeros_hbm.at[pl.ds(0, nvec * 16)], vbb)
        for i in range(CCH):
            vba[i, pl.ds(16, 16)] = onevec
            vbb[i, pl.ds(16, 16)] = onevec
        plsc.subcore_barrier()

        def fill_attr(ab, vb):
            for q in range(nvec):
                av = ab[pl.ds(16 * q, 16)]
                for kk in range(min(16, CCH - 16 * q)):
                    vb[16 * q + kk, pl.ds(0, 16)] = jnp.full(
                        (16,), av[kk], jnp.float32)

        def body(p, carry):
            j0 = 2 * p
            j1 = 2 * p + 1

            @pl.when(p > 0)
            def _():
                pltpu.make_async_copy(vba.at[pl.ds(0, CCH)], dacc.at[rva],
                                      ssema).wait()

            pltpu.async_copy(row2_hbm.at[ch0 + j0], rva, rsem)
            pltpu.async_copy(attr2_hbm.at[ch0 + j0], aba, asem).wait()
            fill_attr(aba, vba)
            pltpu.make_async_copy(row2_hbm.at[ch0 + j0], rva, rsem).wait()
            pltpu.async_copy(vba.at[pl.ds(0, CCH)], dacc.at[rva], ssema,
                             add=True)

            @pl.when(p > 0)
            def _():
                pltpu.make_async_copy(vbb.at[pl.ds(0, CCH)], dacc.at[rvb],
                                      ssemb).wait()

            pltpu.async_copy(row2_hbm.at[ch0 + j1], rvb, rsem)
            pltpu.async_copy(attr2_hbm.at[ch0 + j1], abb, asem).wait()
            fill_attr(abb, vbb)
            pltpu.make_async_copy(row2_hbm.at[ch0 + j1], rvb, rsem).wait()
            pltpu.async_copy(vbb.at[pl.ds(0, CCH)], dacc.at[rvb], ssemb,
                             add=True)
            return carry

        lax.fori_loop(0, nch // 2, body, 0)
        pltpu.make_async_copy(vba.at[pl.ds(0, CCH)], dacc.at[rva], ssema).wait()
        pltpu.make_async_copy(vbb.at[pl.ds(0, CCH)], dacc.at[rvb], ssemb).wait()
        plsc.subcore_barrier()

        @pl.when(s < NS - 1)
        def _():
            pltpu.sync_copy(dacc.at[pl.ds(r0, rpt)], out_hbm.at[pl.ds(ob, rpt)])

        @pl.when(s == NS - 1)
        def _():
            pltpu.sync_copy(
                dacc.at[pl.ds((NS - 1) * rpt, rlast)],
                out_hbm.at[pl.ds(c * n_nodes + (NS - 1) * rpt, rlast)])

    return k


def _make_sc_spmm(n_nodes, n_edges, hdim):
    epw = n_edges // (NC * NS)
    nch = epw // CCH
    rpt, rlast = _row_split(n_nodes)

    @functools.partial(
        pl.kernel,
        out_type=jax.ShapeDtypeStruct((NC * n_nodes, hdim), jnp.float32),
        mesh=_sc_mesh(),
        scratch_types=[
            pltpu.VMEM((nch, CCH), jnp.int32),
            pltpu.VMEM((CCH,), jnp.int32),
            pltpu.VMEM((CCH,), jnp.int32),
            pltpu.VMEM((CCH, hdim), jnp.float32),
            pltpu.VMEM((CCH, hdim), jnp.float32),
            pltpu.VMEM_SHARED((n_nodes, hdim), jnp.float32),
            pltpu.SemaphoreType.DMA,
            pltpu.SemaphoreType.DMA,
            pltpu.SemaphoreType.DMA,
            pltpu.SemaphoreType.DMA,
            pltpu.SemaphoreType.DMA,
            pltpu.SemaphoreType.DMA,
        ],
    )
    def k(col2_hbm, row2_hbm, hw_hbm, zeros_hbm, out_hbm,
          col_all, rva, rvb, bufa, bufb, sacc,
          gsema, gsemb, rsem, rsemb, ssema, ssemb):
        c = lax.axis_index("c")
        s = lax.axis_index("s")
        tid = c * NS + s
        r0 = pl.multiple_of(s * rpt, 8)
        ob = pl.multiple_of(c * n_nodes + s * rpt, 8)
        ch0 = tid * nch
        pltpu.sync_copy(col2_hbm.at[pl.ds(ch0, nch)], col_all)

        @pl.when(s < NS - 1)
        def _():
            pltpu.sync_copy(zeros_hbm.at[pl.ds(r0, rpt)], sacc.at[pl.ds(r0, rpt)])

        @pl.when(s == NS - 1)
        def _():
            pltpu.sync_copy(zeros_hbm.at[pl.ds((NS - 1) * rpt, rlast)],
                            sacc.at[pl.ds((NS - 1) * rpt, rlast)])

        plsc.subcore_barrier()

        def body(p, carry):
            j0 = 2 * p
            j1 = 2 * p + 1

            @pl.when(p > 0)
            def _():
                pltpu.make_async_copy(bufa, sacc.at[rva], ssema).wait()

            pltpu.async_copy(row2_hbm.at[ch0 + j0], rva, rsem)
            pltpu.async_copy(hw_hbm.at[col_all.at[j0]], bufa, gsema)
            pltpu.make_async_copy(row2_hbm.at[ch0 + j0], rva, rsem).wait()
            pltpu.make_async_copy(hw_hbm.at[col_all.at[j0]], bufa, gsema).wait()
            pltpu.async_copy(bufa, sacc.at[rva], ssema, add=True)

            @pl.when(p > 0)
            def _():
                pltpu.make_async_copy(bufb, sacc.at[rvb], ssemb).wait()

            pltpu.async_copy(row2_hbm.at[ch0 + j1], rvb, rsemb)
            pltpu.async_copy(hw_hbm.at[col_all.at[j1]], bufb, gsemb)
            pltpu.make_async_copy(row2_hbm.at[ch0 + j1], rvb, rsemb).wait()
            pltpu.make_async_copy(hw_hbm.at[col_all.at[j1]], bufb, gsemb).wait()
            pltpu.async_copy(bufb, sacc.at[rvb], ssemb, add=True)
            return carry

        lax.fori_loop(0, nch // 2, body, 0)
        pltpu.make_async_copy(bufa, sacc.at[rva], ssema).wait()
        pltpu.make_async_copy(bufb, sacc.at[rvb], ssemb).wait()
        plsc.subcore_barrier()

        @pl.when(s < NS - 1)
        def _():
            pltpu.sync_copy(sacc.at[pl.ds(r0, rpt)], out_hbm.at[pl.ds(ob, rpt)])

        @pl.when(s == NS - 1)
        def _():
            pltpu.sync_copy(
                sacc.at[pl.ds((NS - 1) * rpt, rlast)],
                out_hbm.at[pl.ds(c * n_nodes + (NS - 1) * rpt, rlast)])

    return k


def kernel(x, edge_index, edge_attr, W_msg, b_msg, W_ih, W_hh, b_ih, b_hh):
    n, hdim = x.shape
    n_edges = edge_index.shape[1]
    rounds = W_msg.shape[0]

    nch_all = n_edges // CCH
    row2 = edge_index[0].reshape(nch_all, CCH)
    col2 = edge_index[1].reshape(nch_all, CCH)
    attr2 = jnp.concatenate(
        [edge_attr.reshape(nch_all, CCH),
         jnp.zeros((nch_all, 3), jnp.float32)], axis=1)
    zeros = jnp.zeros((n, hdim), jnp.float32)

    sc_da = _make_sc_da(n, n_edges, hdim)
    sc_spmm = _make_sc_spmm(n, n_edges, hdim)

    da2 = sc_da(row2, attr2, zeros).reshape(NC, n, hdim)

    h = x
    hw = _tc_hw(h, W_msg[0, :hdim, :])
    for t in range(rounds):
        s2 = sc_spmm(col2, row2, hw, zeros).reshape(NC, n, hdim)
        last = t == rounds - 1
        outs = _tc_update(
            h, s2, da2,
            W_msg[t, hdim:2 * hdim, :],
            W_msg[t, 2 * hdim:2 * hdim + 1, :],
            b_msg[t:t + 1],
            W_ih[t], W_hh[t], b_ih[t:t + 1], b_hh[t:t + 1],
            W_msg[(t + 1) % rounds, :hdim, :],
            emit_hw=not last)
        if last:
            h, = outs
        else:
            h, hw = outs
    return h


# speedup vs baseline: 1.0851x; 1.0034x over previous
import functools

import jax
import jax.numpy as jnp
from jax import lax
from jax.experimental import pallas as pl
from jax.experimental.pallas import tpu as pltpu
from jax.experimental.pallas import tpu_sc as plsc

NC = 2
NS = 16
CCH = 125
BN = 2000


def _hw_body(h_ref, wd_ref, out_ref):
    out_ref[...] = jnp.dot(h_ref[...], wd_ref[...],
                           preferred_element_type=jnp.float32)


def _tc_hw(h, wd):
    n, hdim = h.shape
    return pl.pallas_call(
        _hw_body,
        grid=(n // BN,),
        in_specs=[
            pl.BlockSpec((BN, hdim), lambda i: (i, 0)),
            pl.BlockSpec((hdim, hdim), lambda i: (0, 0)),
        ],
        out_specs=pl.BlockSpec((BN, hdim), lambda i: (i, 0)),
        out_shape=jax.ShapeDtypeStruct((n, hdim), jnp.float32),
    )(h, wd)


def _update_body(h_ref, s2_ref, da_ref, wsrc_ref, wattr_ref, bmsg_ref,
                 wih_ref, whh_ref, bih_ref, bhh_ref, wdn_ref,
                 hout_ref, hwout_ref=None):
    hdim = h_ref.shape[1]
    h = h_ref[...]
    s = s2_ref[0] + s2_ref[1]
    deg = da_ref[0, :, 0:1] + da_ref[1, :, 0:1]
    attr_sum = da_ref[0, :, 1:2] + da_ref[1, :, 1:2]
    agg = (s
           + deg * jnp.dot(h, wsrc_ref[...], preferred_element_type=jnp.float32)
           + attr_sum * wattr_ref[...]
           + deg * bmsg_ref[...])
    gi = jnp.dot(agg, wih_ref[...], preferred_element_type=jnp.float32) + bih_ref[...]
    gh = jnp.dot(h, whh_ref[...], preferred_element_type=jnp.float32) + bhh_ref[...]
    r = jax.nn.sigmoid(gi[:, :hdim] + gh[:, :hdim])
    z = jax.nn.sigmoid(gi[:, hdim:2 * hdim] + gh[:, hdim:2 * hdim])
    nn = jnp.tanh(gi[:, 2 * hdim:] + r * gh[:, 2 * hdim:])
    hn = (1.0 - z) * nn + z * h
    hout_ref[...] = hn
    if hwout_ref is not None:
        hwout_ref[...] = jnp.dot(hn, wdn_ref[...],
                                 preferred_element_type=jnp.float32)


def _tc_update(h, s2, da2, wsrc, wattr, bmsg, wih, whh, bih, bhh, wdn,
               emit_hw=True):
    n, hdim = h.shape
    full = lambda a, b: pl.BlockSpec((a, b), lambda i: (0, 0))
    out_specs = [pl.BlockSpec((BN, hdim), lambda i: (i, 0))]
    out_shape = [jax.ShapeDtypeStruct((n, hdim), jnp.float32)]
    if emit_hw:
        out_specs.append(pl.BlockSpec((BN, hdim), lambda i: (i, 0)))
        out_shape.append(jax.ShapeDtypeStruct((n, hdim), jnp.float32))
    return pl.pallas_call(
        _update_body,
        grid=(n // BN,),
        in_specs=[
            pl.BlockSpec((BN, hdim), lambda i: (i, 0)),
            pl.BlockSpec((2, BN, hdim), lambda i: (0, i, 0)),
            pl.BlockSpec((2, BN, hdim), lambda i: (0, i, 0)),
            full(hdim, hdim),
            full(1, hdim),
            full(1, hdim),
            full(hdim, 3 * hdim),
            full(hdim, 3 * hdim),
            full(1, 3 * hdim),
            full(1, 3 * hdim),
            full(hdim, hdim),
        ],
        out_specs=out_specs,
        out_shape=out_shape,
    )(h, s2, da2, wsrc, wattr, bmsg, wih, whh, bih, bhh, wdn)


def _sc_mesh():
    return plsc.VectorSubcoreMesh(
        core_axis_name="c", subcore_axis_name="s", num_cores=NC, num_subcores=NS)


def _row_split(n_nodes):
    rpt = -(-(n_nodes // NS) // 8) * 8
    rlast = n_nodes - (NS - 1) * rpt
    return rpt, rlast


def _make_sc_da(n_nodes, n_edges, hdim):
    epw = n_edges // (NC * NS)
    nch = epw // CCH
    rpt, rlast = _row_split(n_nodes)
    nvec = -(-CCH // 16)

    @functools.partial(
        pl.kernel,
        out_type=jax.ShapeDtypeStruct((NC * n_nodes, hdim), jnp.float32),
        mesh=_sc_mesh(),
        scratch_types=[
            pltpu.VMEM((CCH,), jnp.int32),
            pltpu.VMEM((CCH,), jnp.int32),
            pltpu.VMEM((CCH,), jnp.float32),
            pltpu.VMEM((CCH,), jnp.float32),
            pltpu.VMEM((nvec * 16, hdim), jnp.float32),
            pltpu.VMEM((nvec * 16, hdim), jnp.float32),
            pltpu.VMEM_SHARED((n_nodes, hdim), jnp.float32),
            pltpu.SemaphoreType.DMA,
            pltpu.SemaphoreType.DMA,
            pltpu.SemaphoreType.DMA,
            pltpu.SemaphoreType.DMA,
        ],
    )
    def k(row2_hbm, attr2_hbm, zeros_hbm, out_hbm,
          rva, rvb, aba, abb, vba, vbb, dacc, asem, rsem, ssema, ssemb):
        c = lax.axis_index("c")
        s = lax.axis_index("s")
        tid = c * NS + s
        r0 = pl.multiple_of(s * rpt, 8)
        ob = pl.multiple_of(c * n_nodes + s * rpt, 8)
        ch0 = tid * nch

        @pl.when(s < NS - 1)
        def _():
            pltpu.sync_copy(zeros_hbm.at[pl.ds(r0, rpt)], dacc.at[pl.ds(r0, rpt)])

        @pl.when(s == NS - 1)
        def _():
            pltpu.sync_copy(zeros_hbm.at[pl.ds((NS - 1) * rpt, rlast)],
                            dacc.at[pl.ds((NS - 1) * rpt, rlast)])

        lanes = lax.iota(jnp.int32, 16)
        pltpu.sync_copy(zeros_hbm.at[pl.ds(0, nvec * 16)], vba)
        pltpu.sync_copy(zeros_hbm.at[pl.ds(0, nvec * 16)], vbb)
        plsc.subcore_barrier()

        def fill_attr(ab, vb):
            def fill16(base):
                av = ab[pl.ds(base, 16)]
                for kk in range(16):
                    vb[base + kk, pl.ds(0, 16)] = jnp.where(
                        lanes == 0, jnp.float32(1.0),
                        jnp.where(lanes == 1, av[kk], jnp.float32(0.0)))
            for q in range(CCH // 16):
                fill16(16 * q)
            if CCH % 16:
                fill16(CCH - 16)

        def body(p, carry):
            j0 = 2 * p
            j1 = 2 * p + 1

            @pl.when(p > 0)
            def _():
                pltpu.make_async_copy(vba.at[pl.ds(0, CCH)], dacc.at[rva],
                                      ssema).wait()

            pltpu.async_copy(row2_hbm.at[ch0 + j0], rva, rsem)
            pltpu.async_copy(attr2_hbm.at[ch0 + j0], aba, asem).wait()
            fill_attr(aba, vba)
            pltpu.make_async_copy(row2_hbm.at[ch0 + j0], rva, rsem).wait()
            pltpu.async_copy(vba.at[pl.ds(0, CCH)], dacc.at[rva], ssema,
                             add=True)

            @pl.when(p > 0)
            def _():
                pltpu.make_async_copy(vbb.at[pl.ds(0, CCH)], dacc.at[rvb],
                                      ssemb).wait()

            pltpu.async_copy(row2_hbm.at[ch0 + j1], rvb, rsem)
            pltpu.async_copy(attr2_hbm.at[ch0 + j1], abb, asem).wait()
            fill_attr(abb, vbb)
            pltpu.make_async_copy(row2_hbm.at[ch0 + j1], rvb, rsem).wait()
            pltpu.async_copy(vbb.at[pl.ds(0, CCH)], dacc.at[rvb], ssemb,
                             add=True)
            return carry

        lax.fori_loop(0, nch // 2, body, 0)
        pltpu.make_async_copy(vba.at[pl.ds(0, CCH)], dacc.at[rva], ssema).wait()
        pltpu.make_async_copy(vbb.at[pl.ds(0, CCH)], dacc.at[rvb], ssemb).wait()
        plsc.subcore_barrier()

        @pl.when(s < NS - 1)
        def _():
            pltpu.sync_copy(dacc.at[pl.ds(r0, rpt)], out_hbm.at[pl.ds(ob, rpt)])

        @pl.when(s == NS - 1)
        def _():
            pltpu.sync_copy(
                dacc.at[pl.ds((NS - 1) * rpt, rlast)],
                out_hbm.at[pl.ds(c * n_nodes + (NS - 1) * rpt, rlast)])

    return k


def _make_sc_spmm(n_nodes, n_edges, hdim):
    epw = n_edges // (NC * NS)
    nch = epw // CCH
    rpt, rlast = _row_split(n_nodes)

    @functools.partial(
        pl.kernel,
        out_type=jax.ShapeDtypeStruct((NC * n_nodes, hdim), jnp.float32),
        mesh=_sc_mesh(),
        scratch_types=[
            pltpu.VMEM((nch, CCH), jnp.int32),
            pltpu.VMEM((CCH,), jnp.int32),
            pltpu.VMEM((CCH,), jnp.int32),
            pltpu.VMEM((CCH, hdim), jnp.float32),
            pltpu.VMEM((CCH, hdim), jnp.float32),
            pltpu.VMEM_SHARED((n_nodes, hdim), jnp.float32),
            pltpu.SemaphoreType.DMA,
            pltpu.SemaphoreType.DMA,
            pltpu.SemaphoreType.DMA,
            pltpu.SemaphoreType.DMA,
            pltpu.SemaphoreType.DMA,
            pltpu.SemaphoreType.DMA,
        ],
    )
    def k(col2_hbm, row2_hbm, hw_hbm, zeros_hbm, out_hbm,
          col_all, rva, rvb, bufa, bufb, sacc,
          gsema, gsemb, rsem, rsemb, ssema, ssemb):
        c = lax.axis_index("c")
        s = lax.axis_index("s")
        tid = c * NS + s
        r0 = pl.multiple_of(s * rpt, 8)
        ob = pl.multiple_of(c * n_nodes + s * rpt, 8)
        ch0 = tid * nch
        pltpu.sync_copy(col2_hbm.at[pl.ds(ch0, nch)], col_all)

        @pl.when(s < NS - 1)
        def _():
            pltpu.sync_copy(zeros_hbm.at[pl.ds(r0, rpt)], sacc.at[pl.ds(r0, rpt)])

        @pl.when(s == NS - 1)
        def _():
            pltpu.sync_copy(zeros_hbm.at[pl.ds((NS - 1) * rpt, rlast)],
                            sacc.at[pl.ds((NS - 1) * rpt, rlast)])

        plsc.subcore_barrier()

        def body(p, carry):
            j0 = 2 * p
            j1 = 2 * p + 1

            @pl.when(p > 0)
            def _():
                pltpu.make_async_copy(bufa, sacc.at[rva], ssema).wait()

            pltpu.async_copy(row2_hbm.at[ch0 + j0], rva, rsem)
            pltpu.async_copy(hw_hbm.at[col_all.at[j0]], bufa, gsema)
            pltpu.make_async_copy(row2_hbm.at[ch0 + j0], rva, rsem).wait()
            pltpu.make_async_copy(hw_hbm.at[col_all.at[j0]], bufa, gsema).wait()
            pltpu.async_copy(bufa, sacc.at[rva], ssema, add=True)

            @pl.when(p > 0)
            def _():
                pltpu.make_async_copy(bufb, sacc.at[rvb], ssemb).wait()

            pltpu.async_copy(row2_hbm.at[ch0 + j1], rvb, rsemb)
            pltpu.async_copy(hw_hbm.at[col_all.at[j1]], bufb, gsemb)
            pltpu.make_async_copy(row2_hbm.at[ch0 + j1], rvb, rsemb).wait()
            pltpu.make_async_copy(hw_hbm.at[col_all.at[j1]], bufb, gsemb).wait()
            pltpu.async_copy(bufb, sacc.at[rvb], ssemb, add=True)
            return carry

        lax.fori_loop(0, nch // 2, body, 0)
        pltpu.make_async_copy(bufa, sacc.at[rva], ssema).wait()
        pltpu.make_async_copy(bufb, sacc.at[rvb], ssemb).wait()
        plsc.subcore_barrier()

        @pl.when(s < NS - 1)
        def _():
            pltpu.sync_copy(sacc.at[pl.ds(r0, rpt)], out_hbm.at[pl.ds(ob, rpt)])

        @pl.when(s == NS - 1)
        def _():
            pltpu.sync_copy(
                sacc.at[pl.ds((NS - 1) * rpt, rlast)],
                out_hbm.at[pl.ds(c * n_nodes + (NS - 1) * rpt, rlast)])

    return k


def kernel(x, edge_index, edge_attr, W_msg, b_msg, W_ih, W_hh, b_ih, b_hh):
    n, hdim = x.shape
    n_edges = edge_index.shape[1]
    rounds = W_msg.shape[0]

    nch_all = n_edges // CCH
    row2 = edge_index[0].reshape(nch_all, CCH)
    col2 = edge_index[1].reshape(nch_all, CCH)
    attr2 = edge_attr.reshape(nch_all, CCH)
    zeros = jnp.zeros((n, hdim), jnp.float32)

    sc_da = _make_sc_da(n, n_edges, hdim)
    sc_spmm = _make_sc_spmm(n, n_edges, hdim)

    da2 = sc_da(row2, attr2, zeros).reshape(NC, n, hdim)

    h = x
    hw = _tc_hw(h, W_msg[0, :hdim, :])
    for t in range(rounds):
        s2 = sc_spmm(col2, row2, hw, zeros).reshape(NC, n, hdim)
        last = t == rounds - 1
        outs = _tc_update(
            h, s2, da2,
            W_msg[t, hdim:2 * hdim, :],
            W_msg[t, 2 * hdim:2 * hdim + 1, :],
            b_msg[t:t + 1],
            W_ih[t], W_hh[t], b_ih[t:t + 1], b_hh[t:t + 1],
            W_msg[(t + 1) % rounds, :hdim, :],
            emit_hw=not last)
        if last:
            h, = outs
        else:
            h, hw = outs
    return h
